# baseline (speedup 1.0000x reference)
import jax
import jax.numpy as jnp
from jax.experimental import pallas as pl
from jax.experimental.pallas import tpu as pltpu


def _copy_kernel(in_ref, out_ref, sem):
    copy = pltpu.make_async_copy(in_ref, out_ref, sem)
    copy.start()
    copy.wait()


def kernel(xyz, reconstruction, neighborsMatrix, numNeighbors, weightMatrix, arapWeight):
    n, d = reconstruction.shape
    flat = reconstruction.reshape(-1)
    out = pl.pallas_call(
        _copy_kernel,
        out_shape=jax.ShapeDtypeStruct(flat.shape, flat.dtype),
        in_specs=[pl.BlockSpec(memory_space=pltpu.MemorySpace.HBM)],
        out_specs=pl.BlockSpec(memory_space=pltpu.MemorySpace.HBM),
        scratch_shapes=[pltpu.SemaphoreType.DMA],
    )(flat)
    return out.reshape(n, d)


# speedup vs baseline: 1.2651x; 1.2651x over previous
import jax
import jax.numpy as jnp
from jax.experimental import pallas as pl
from jax.experimental.pallas import tpu as pltpu


def _identity_kernel(in_ref, out_ref):
    del in_ref, out_ref


def kernel(xyz, reconstruction, neighborsMatrix, numNeighbors, weightMatrix, arapWeight):
    n, d = reconstruction.shape
    flat = reconstruction.reshape(-1)
    out = pl.pallas_call(
        _identity_kernel,
        out_shape=jax.ShapeDtypeStruct(flat.shape, flat.dtype),
        in_specs=[pl.BlockSpec(memory_space=pltpu.MemorySpace.HBM)],
        out_specs=pl.BlockSpec(memory_space=pltpu.MemorySpace.HBM),
        input_output_aliases={0: 0},
    )(flat)
    return out.reshape(n, d)


# speedup vs baseline: 1.7500x; 1.3833x over previous
import jax
import jax.numpy as jnp
from jax.experimental import pallas as pl


def _copy_kernel(in_ref, out_ref):
    out_ref[...] = in_ref[...]


def kernel(xyz, reconstruction, neighborsMatrix, numNeighbors, weightMatrix, arapWeight):
    n, d = reconstruction.shape
    blk = 2000
    return pl.pallas_call(
        _copy_kernel,
        grid=(pl.cdiv(n, blk),),
        in_specs=[pl.BlockSpec((blk, d), lambda i: (i, 0))],
        out_specs=pl.BlockSpec((blk, d), lambda i: (i, 0)),
        out_shape=jax.ShapeDtypeStruct(reconstruction.shape, reconstruction.dtype),
    )(reconstruction)


# speedup vs baseline: 2.1423x; 1.2241x over previous
import jax
import jax.numpy as jnp
from jax.experimental import pallas as pl


def _copy_kernel(in_ref, out_ref):
    out_ref[...] = in_ref[...]


def kernel(xyz, reconstruction, neighborsMatrix, numNeighbors, weightMatrix, arapWeight):
    n, d = reconstruction.shape
    blk = 20000
    return pl.pallas_call(
        _copy_kernel,
        grid=(pl.cdiv(n, blk),),
        in_specs=[pl.BlockSpec((blk, d), lambda i: (i, 0))],
        out_specs=pl.BlockSpec((blk, d), lambda i: (i, 0)),
        out_shape=jax.ShapeDtypeStruct(reconstruction.shape, reconstruction.dtype),
    )(reconstruction)
